# baseline (speedup 1.0000x reference)
import functools

import jax
import jax.numpy as jnp
from jax import lax
from jax.experimental import pallas as pl
from jax.experimental.pallas import tpu as pltpu
from jax.experimental.pallas import tpu_sc as plsc

N = 10000
E = 320000
D = 128
H = 128
NUM_MASK = N // 2

NC = 2
NS = 16
NW = NC * NS

CHUNK = 128
CPT = 80
EPAD = NW * CPT * CHUNK
MCPT = 2
MASKPAD = NW * MCPT * CHUNK
SINK_ROWS = 768
NSINK = N + SINK_ROWS

def _per_tile_copy(sid, nrows, src_fn, dst_fn):
    base = (nrows // NS) // 8 * 8
    last = nrows - base * (NS - 1)
    r0 = pl.multiple_of(sid * base, 8)

    @pl.when(sid < NS - 1)
    def _():
        pltpu.sync_copy(src_fn(r0, base), dst_fn(r0, base))

    @pl.when(sid == NS - 1)
    def _():
        r1 = base * (NS - 1)
        pltpu.sync_copy(src_fn(r1, last), dst_fn(r1, last))


def _make_hist(cpt):
    def body(idx3_hbm, ones_hbm, z_hbm, out_hbm, idxb, ones_v, acc_sh):
        cid = lax.axis_index("c")
        sid = lax.axis_index("s")
        wid = sid * NC + cid

        pltpu.sync_copy(ones_hbm, ones_v)
        pltpu.sync_copy(idx3_hbm.at[wid], idxb)
        _per_tile_copy(sid, NSINK,
                       lambda o, s: z_hbm.at[pl.ds(o, s)],
                       lambda o, s: acc_sh.at[pl.ds(o, s)])
        plsc.subcore_barrier()

        def bd(c, carry):
            pltpu.sync_copy(ones_v, acc_sh.at[idxb.at[c]], add=True)
            return carry

        lax.fori_loop(0, cpt, bd, 0)
        plsc.subcore_barrier()

        _per_tile_copy(sid, N,
                       lambda o, s: acc_sh.at[pl.ds(o, s)],
                       lambda o, s: out_hbm.at[cid, pl.ds(o, s)])

    return pl.kernel(
        body,
        out_type=jax.ShapeDtypeStruct((NC, N, H), jnp.float32),
        mesh=plsc.VectorSubcoreMesh(core_axis_name="c", subcore_axis_name="s",
                                    num_cores=NC, num_subcores=NS),
        scratch_types=[
            pltpu.VMEM((cpt, CHUNK), jnp.int32),
            pltpu.VMEM((CHUNK, H), jnp.float32),
            pltpu.VMEM_SHARED((NSINK, H), jnp.float32),
        ],
    )


_sc_deg_hist = _make_hist(CPT)
_sc_mask_hist = _make_hist(MCPT)


def _sc_agg_body(y_hbm, src3_hbm, dst3_hbm, z_hbm, out_hbm,
                 srcb, dstb, rows_a, rows_b, agg_sh, gsem_a, gsem_b):
    cid = lax.axis_index("c")
    sid = lax.axis_index("s")
    wid = sid * NC + cid

    _per_tile_copy(sid, NSINK,
                   lambda o, s: z_hbm.at[pl.ds(o, s)],
                   lambda o, s: agg_sh.at[pl.ds(o, s)])
    plsc.subcore_barrier()

    PH = CPT // 2

    for p in range(2):
        pltpu.sync_copy(src3_hbm.at[wid, pl.ds(p * PH, PH)], srcb)
        pltpu.sync_copy(dst3_hbm.at[wid, pl.ds(p * PH, PH)], dstb)
        pltpu.async_copy(y_hbm.at[srcb.at[0]], rows_a, gsem_a)

        def pair(i, carry):
            ca = 2 * i
            cb = ca + 1
            pltpu.async_copy(y_hbm.at[srcb.at[cb]], rows_b, gsem_b)
            pltpu.make_async_copy(y_hbm.at[srcb.at[ca]], rows_a, gsem_a).wait()
            pltpu.sync_copy(rows_a, agg_sh.at[dstb.at[ca]], add=True)

            @pl.when(i < PH // 2 - 1)
            def _():
                pltpu.async_copy(y_hbm.at[srcb.at[ca + 2]], rows_a, gsem_a)

            pltpu.make_async_copy(y_hbm.at[srcb.at[cb]], rows_b, gsem_b).wait()
            pltpu.sync_copy(rows_b, agg_sh.at[dstb.at[cb]], add=True)
            return carry

        lax.fori_loop(0, PH // 2, pair, 0)

    plsc.subcore_barrier()

    _per_tile_copy(sid, N,
                   lambda o, s: agg_sh.at[pl.ds(o, s)],
                   lambda o, s: out_hbm.at[cid, pl.ds(o, s)])


_sc_agg = pl.kernel(
    _sc_agg_body,
    out_type=jax.ShapeDtypeStruct((NC, N, H), jnp.float32),
    mesh=plsc.VectorSubcoreMesh(core_axis_name="c", subcore_axis_name="s", num_cores=NC, num_subcores=NS),
    scratch_types=[
        pltpu.VMEM((CPT // 2, CHUNK), jnp.int32),
        pltpu.VMEM((CPT // 2, CHUNK), jnp.int32),
        pltpu.VMEM((CHUNK, H), jnp.float32),
        pltpu.VMEM((CHUNK, H), jnp.float32),
        pltpu.VMEM_SHARED((NSINK, H), jnp.float32),
        pltpu.SemaphoreType.DMA,
        pltpu.SemaphoreType.DMA,
    ],
)


BN = 1000
GRID = N // BN


def _tca_body(x_ref, m0_ref, m1_ref, tok_ref, w_ref, b_ref, o_ref):
    m = m0_ref[0][:, 0:1] + m1_ref[0][:, 0:1]
    xm = x_ref[...] * (1.0 - m) + m * tok_ref[...]
    o_ref[...] = (jnp.dot(xm, w_ref[...], preferred_element_type=jnp.float32)
                  + b_ref[...])


_tca = pl.pallas_call(
    _tca_body,
    grid=(GRID,),
    in_specs=[
        pl.BlockSpec((BN, D), lambda i: (i, 0)),
        pl.BlockSpec((1, BN, 128), lambda i: (0, i, 0)),
        pl.BlockSpec((1, BN, 128), lambda i: (1, i, 0)),
        pl.BlockSpec((1, D), lambda i: (0, 0)),
        pl.BlockSpec((D, H), lambda i: (0, 0)),
        pl.BlockSpec((1, H), lambda i: (0, 0)),
    ],
    out_specs=pl.BlockSpec((BN, H), lambda i: (i, 0)),
    out_shape=jax.ShapeDtypeStruct((N, H), jnp.float32),
)


def _tcb_body(a0_ref, a1_ref, d0_ref, d1_ref, w_ref, b_ref, h1_ref, z2_ref):
    agg = a0_ref[0] + a1_ref[0]
    deg = jnp.maximum(d0_ref[0][:, 0:1] + d1_ref[0][:, 0:1], 1.0)
    h1 = jnp.maximum(agg / deg, 0.0)
    h1_ref[...] = h1
    z2_ref[...] = (jnp.dot(h1, w_ref[...], preferred_element_type=jnp.float32)
                   + b_ref[...])


_tcb = pl.pallas_call(
    _tcb_body,
    grid=(GRID,),
    in_specs=[
        pl.BlockSpec((1, BN, H), lambda i: (0, i, 0)),
        pl.BlockSpec((1, BN, H), lambda i: (1, i, 0)),
        pl.BlockSpec((1, BN, 128), lambda i: (0, i, 0)),
        pl.BlockSpec((1, BN, 128), lambda i: (1, i, 0)),
        pl.BlockSpec((H, H), lambda i: (0, 0)),
        pl.BlockSpec((1, H), lambda i: (0, 0)),
    ],
    out_specs=[
        pl.BlockSpec((BN, H), lambda i: (i, 0)),
        pl.BlockSpec((BN, H), lambda i: (i, 0)),
    ],
    out_shape=[
        jax.ShapeDtypeStruct((N, H), jnp.float32),
        jax.ShapeDtypeStruct((N, H), jnp.float32),
    ],
)


def _tcc_body(a0_ref, a1_ref, d0_ref, d1_ref, h1_ref, m0_ref, m1_ref,
              we1_ref, we2_ref, wd_ref, bd_ref, z3_ref):
    agg = a0_ref[0] + a1_ref[0]
    deg = jnp.maximum(d0_ref[0][:, 0:1] + d1_ref[0][:, 0:1], 1.0)
    h2 = jnp.maximum(agg / deg, 0.0)
    rep = (jnp.dot(h1_ref[...], we1_ref[...], preferred_element_type=jnp.float32)
           + jnp.dot(h2, we2_ref[...], preferred_element_type=jnp.float32))
    m = m0_ref[0][:, 0:1] + m1_ref[0][:, 0:1]
    rep = rep * (1.0 - m)
    z3_ref[...] = (jnp.dot(rep, wd_ref[...], preferred_element_type=jnp.float32)
                   + bd_ref[...])


_tcc = pl.pallas_call(
    _tcc_body,
    grid=(GRID,),
    in_specs=[
        pl.BlockSpec((1, BN, H), lambda i: (0, i, 0)),
        pl.BlockSpec((1, BN, H), lambda i: (1, i, 0)),
        pl.BlockSpec((1, BN, 128), lambda i: (0, i, 0)),
        pl.BlockSpec((1, BN, 128), lambda i: (1, i, 0)),
        pl.BlockSpec((BN, H), lambda i: (i, 0)),
        pl.BlockSpec((1, BN, 128), lambda i: (0, i, 0)),
        pl.BlockSpec((1, BN, 128), lambda i: (1, i, 0)),
        pl.BlockSpec((H, H), lambda i: (0, 0)),
        pl.BlockSpec((H, H), lambda i: (0, 0)),
        pl.BlockSpec((H, D), lambda i: (0, 0)),
        pl.BlockSpec((1, D), lambda i: (0, 0)),
    ],
    out_specs=pl.BlockSpec((BN, D), lambda i: (i, 0)),
    out_shape=jax.ShapeDtypeStruct((N, D), jnp.float32),
)


def _tcd_body(a0_ref, a1_ref, d0_ref, d1_ref, m0_ref, m1_ref, x_ref, o_ref):
    i = pl.program_id(0)
    agg = a0_ref[0] + a1_ref[0]
    deg = jnp.maximum(d0_ref[0][:, 0:1] + d1_ref[0][:, 0:1], 1.0)
    recon = agg / deg
    xb = x_ref[...]
    xn = xb / (jnp.sqrt(jnp.sum(xb * xb, axis=-1, keepdims=True)) + 1e-8)
    rn = recon / (jnp.sqrt(jnp.sum(recon * recon, axis=-1, keepdims=True)) + 1e-8)
    cos = jnp.sum(xn * rn, axis=-1)
    mcol = m0_ref[0][:, 0] + m1_ref[0][:, 0]
    part = jnp.sum(mcol * (1.0 - cos) ** 2) * (1.0 / NUM_MASK)

    @pl.when(i == 0)
    def _():
        o_ref[...] = jnp.zeros((1, 1), jnp.float32)

    o_ref[...] = o_ref[...] + part


_tcd = pl.pallas_call(
    _tcd_body,
    grid=(GRID,),
    in_specs=[
        pl.BlockSpec((1, BN, D), lambda i: (0, i, 0)),
        pl.BlockSpec((1, BN, D), lambda i: (1, i, 0)),
        pl.BlockSpec((1, BN, 128), lambda i: (0, i, 0)),
        pl.BlockSpec((1, BN, 128), lambda i: (1, i, 0)),
        pl.BlockSpec((1, BN, 128), lambda i: (0, i, 0)),
        pl.BlockSpec((1, BN, 128), lambda i: (1, i, 0)),
        pl.BlockSpec((BN, D), lambda i: (i, 0)),
    ],
    out_specs=pl.BlockSpec((1, 1), lambda i: (0, 0)),
    out_shape=jax.ShapeDtypeStruct((1, 1), jnp.float32),
)


def kernel(x, edge_index, mask_nodes, enc_mask_token, W1, b1, W2, b2, W_e2d, Wd, bd):
    src = edge_index[0]
    dst = edge_index[1]
    src3 = jnp.concatenate(
        [src, jnp.zeros((EPAD - E,), jnp.int32)]).reshape(NW, CPT, CHUNK)
    dst3 = jnp.concatenate(
        [dst, N + (jnp.arange(EPAD - E, dtype=jnp.int32) % SINK_ROWS)]
    ).reshape(NW, CPT, CHUNK)
    mn3 = jnp.concatenate(
        [mask_nodes.astype(jnp.int32),
         N + (jnp.arange(MASKPAD - NUM_MASK, dtype=jnp.int32) % SINK_ROWS)]
    ).reshape(NW, MCPT, CHUNK)
    ones128 = jnp.ones((CHUNK, H), jnp.float32)
    zsink = jnp.zeros((NSINK, H), jnp.float32)
    b1r = b1.reshape(1, H)
    b2r = b2.reshape(1, H)
    bdr = bd.reshape(1, D)
    we1 = W_e2d[:H]
    we2 = W_e2d[H:]

    degp = _sc_deg_hist(dst3, ones128, zsink)
    maskp = _sc_mask_hist(mn3, ones128, zsink)
    z1 = _tca(x, maskp, maskp, enc_mask_token, W1, b1r)
    a1 = _sc_agg(z1, src3, dst3, zsink)
    h1, z2 = _tcb(a1, a1, degp, degp, W2, b2r)
    a2 = _sc_agg(z2, src3, dst3, zsink)
    z3 = _tcc(a2, a2, degp, degp, h1, maskp, maskp, we1, we2, Wd, bdr)
    a3 = _sc_agg(z3, src3, dst3, zsink)
    lossm = _tcd(a3, a3, degp, degp, maskp, maskp, x)
    return lossm[0, 0]


# speedup vs baseline: 3.7530x; 3.7530x over previous
import functools

import jax
import jax.numpy as jnp
from jax import lax
from jax.experimental import pallas as pl
from jax.experimental.pallas import tpu as pltpu
from jax.experimental.pallas import tpu_sc as plsc

N = 10000
E = 320000
D = 128
H = 128
NUM_MASK = N // 2

NC = 2
NS = 16
NW = NC * NS

CHUNK = 128
CPT = 80
EPAD = NW * CPT * CHUNK
MCPT = 2
MASKPAD = NW * MCPT * CHUNK
SINK_ROWS = 768
NSINK = N + SINK_ROWS

def _per_tile_copy(sid, nrows, src_fn, dst_fn):
    base = (nrows // NS) // 8 * 8
    last = nrows - base * (NS - 1)
    r0 = pl.multiple_of(sid * base, 8)

    @pl.when(sid < NS - 1)
    def _():
        pltpu.sync_copy(src_fn(r0, base), dst_fn(r0, base))

    @pl.when(sid == NS - 1)
    def _():
        r1 = base * (NS - 1)
        pltpu.sync_copy(src_fn(r1, last), dst_fn(r1, last))


def _make_hist(cpt):
    def body(idx3_hbm, ones_hbm, z_hbm, out_hbm, idxb, ones_v, acc_sh):
        cid = lax.axis_index("c")
        sid = lax.axis_index("s")
        wid = sid * NC + cid

        pltpu.sync_copy(ones_hbm, ones_v)
        pltpu.sync_copy(idx3_hbm.at[wid], idxb)
        _per_tile_copy(sid, NSINK,
                       lambda o, s: z_hbm.at[pl.ds(o, s)],
                       lambda o, s: acc_sh.at[pl.ds(o, s)])
        plsc.subcore_barrier()

        def bd(c, carry):
            pltpu.sync_copy(ones_v, acc_sh.at[idxb.at[c]], add=True)
            return carry

        lax.fori_loop(0, cpt, bd, 0)
        plsc.subcore_barrier()

        _per_tile_copy(sid, N,
                       lambda o, s: acc_sh.at[pl.ds(o, s)],
                       lambda o, s: out_hbm.at[cid, pl.ds(o, s)])

    return pl.kernel(
        body,
        out_type=jax.ShapeDtypeStruct((NC, N, H), jnp.float32),
        mesh=plsc.VectorSubcoreMesh(core_axis_name="c", subcore_axis_name="s",
                                    num_cores=NC, num_subcores=NS),
        scratch_types=[
            pltpu.VMEM((cpt, CHUNK), jnp.int32),
            pltpu.VMEM((CHUNK, H), jnp.float32),
            pltpu.VMEM_SHARED((NSINK, H), jnp.float32),
        ],
    )


_sc_deg_hist = _make_hist(CPT)
_sc_mask_hist = _make_hist(MCPT)


def _sc_agg_body(y_hbm, src3_hbm, dst3_hbm, z_hbm, out_hbm,
                 srcb, dstb, rows_a, rows_b, agg_sh, gsem_a, gsem_b):
    cid = lax.axis_index("c")
    sid = lax.axis_index("s")
    wid = sid * NC + cid

    _per_tile_copy(sid, NSINK,
                   lambda o, s: z_hbm.at[pl.ds(o, s)],
                   lambda o, s: agg_sh.at[pl.ds(o, s)])
    plsc.subcore_barrier()

    PH = CPT // 2

    for p in range(2):
        pltpu.sync_copy(src3_hbm.at[wid, pl.ds(p * PH, PH)], srcb)
        pltpu.sync_copy(dst3_hbm.at[wid, pl.ds(p * PH, PH)], dstb)
        pltpu.async_copy(y_hbm.at[srcb.at[0]], rows_a, gsem_a)

        def pair(i, carry):
            ca = 2 * i
            cb = ca + 1
            pltpu.async_copy(y_hbm.at[srcb.at[cb]], rows_b, gsem_b)
            pltpu.make_async_copy(y_hbm.at[srcb.at[ca]], rows_a, gsem_a).wait()
            pltpu.sync_copy(rows_a, agg_sh.at[dstb.at[ca]], add=True)

            @pl.when(i < PH // 2 - 1)
            def _():
                pltpu.async_copy(y_hbm.at[srcb.at[ca + 2]], rows_a, gsem_a)

            pltpu.make_async_copy(y_hbm.at[srcb.at[cb]], rows_b, gsem_b).wait()
            pltpu.sync_copy(rows_b, agg_sh.at[dstb.at[cb]], add=True)
            return carry

        lax.fori_loop(0, PH // 2, pair, 0)

    plsc.subcore_barrier()

    _per_tile_copy(sid, N,
                   lambda o, s: agg_sh.at[pl.ds(o, s)],
                   lambda o, s: out_hbm.at[cid, pl.ds(o, s)])


_sc_agg = pl.kernel(
    _sc_agg_body,
    out_type=jax.ShapeDtypeStruct((NC, N, H), jnp.float32),
    mesh=plsc.VectorSubcoreMesh(core_axis_name="c", subcore_axis_name="s", num_cores=NC, num_subcores=NS),
    scratch_types=[
        pltpu.VMEM((CPT // 2, CHUNK), jnp.int32),
        pltpu.VMEM((CPT // 2, CHUNK), jnp.int32),
        pltpu.VMEM((CHUNK, H), jnp.float32),
        pltpu.VMEM((CHUNK, H), jnp.float32),
        pltpu.VMEM_SHARED((NSINK, H), jnp.float32),
        pltpu.SemaphoreType.DMA,
        pltpu.SemaphoreType.DMA,
    ],
)


BN = 1000
GRID = N // BN


def _tca_body(x_ref, m0_ref, m1_ref, tok_ref, w_ref, b_ref, o_ref):
    m = m0_ref[0][:, 0:1] + m1_ref[0][:, 0:1]
    xm = x_ref[...] * (1.0 - m) + m * tok_ref[...]
    o_ref[...] = (jnp.dot(xm, w_ref[...], preferred_element_type=jnp.float32)
                  + b_ref[...])


_tca = pl.pallas_call(
    _tca_body,
    grid=(GRID,),
    in_specs=[
        pl.BlockSpec((BN, D), lambda i: (i, 0)),
        pl.BlockSpec((1, BN, 128), lambda i: (0, i, 0)),
        pl.BlockSpec((1, BN, 128), lambda i: (1, i, 0)),
        pl.BlockSpec((1, D), lambda i: (0, 0)),
        pl.BlockSpec((D, H), lambda i: (0, 0)),
        pl.BlockSpec((1, H), lambda i: (0, 0)),
    ],
    out_specs=pl.BlockSpec((BN, H), lambda i: (i, 0)),
    out_shape=jax.ShapeDtypeStruct((N, H), jnp.float32),
)


def _tcb_body(a0_ref, a1_ref, d0_ref, d1_ref, w_ref, b_ref, h1_ref, z2_ref):
    agg = a0_ref[0] + a1_ref[0]
    deg = jnp.maximum(d0_ref[0][:, 0:1] + d1_ref[0][:, 0:1], 1.0)
    h1 = jnp.maximum(agg / deg, 0.0)
    h1_ref[...] = h1
    z2_ref[...] = (jnp.dot(h1, w_ref[...], preferred_element_type=jnp.float32)
                   + b_ref[...])


_tcb = pl.pallas_call(
    _tcb_body,
    grid=(GRID,),
    in_specs=[
        pl.BlockSpec((1, BN, H), lambda i: (0, i, 0)),
        pl.BlockSpec((1, BN, H), lambda i: (1, i, 0)),
        pl.BlockSpec((1, BN, 128), lambda i: (0, i, 0)),
        pl.BlockSpec((1, BN, 128), lambda i: (1, i, 0)),
        pl.BlockSpec((H, H), lambda i: (0, 0)),
        pl.BlockSpec((1, H), lambda i: (0, 0)),
    ],
    out_specs=[
        pl.BlockSpec((BN, H), lambda i: (i, 0)),
        pl.BlockSpec((BN, H), lambda i: (i, 0)),
    ],
    out_shape=[
        jax.ShapeDtypeStruct((N, H), jnp.float32),
        jax.ShapeDtypeStruct((N, H), jnp.float32),
    ],
)


def _tcc_body(a0_ref, a1_ref, d0_ref, d1_ref, h1_ref, m0_ref, m1_ref,
              we1_ref, we2_ref, wd_ref, bd_ref, z3_ref):
    agg = a0_ref[0] + a1_ref[0]
    deg = jnp.maximum(d0_ref[0][:, 0:1] + d1_ref[0][:, 0:1], 1.0)
    h2 = jnp.maximum(agg / deg, 0.0)
    rep = (jnp.dot(h1_ref[...], we1_ref[...], preferred_element_type=jnp.float32)
           + jnp.dot(h2, we2_ref[...], preferred_element_type=jnp.float32))
    m = m0_ref[0][:, 0:1] + m1_ref[0][:, 0:1]
    rep = rep * (1.0 - m)
    z3_ref[...] = (jnp.dot(rep, wd_ref[...], preferred_element_type=jnp.float32)
                   + bd_ref[...])


_tcc = pl.pallas_call(
    _tcc_body,
    grid=(GRID,),
    in_specs=[
        pl.BlockSpec((1, BN, H), lambda i: (0, i, 0)),
        pl.BlockSpec((1, BN, H), lambda i: (1, i, 0)),
        pl.BlockSpec((1, BN, 128), lambda i: (0, i, 0)),
        pl.BlockSpec((1, BN, 128), lambda i: (1, i, 0)),
        pl.BlockSpec((BN, H), lambda i: (i, 0)),
        pl.BlockSpec((1, BN, 128), lambda i: (0, i, 0)),
        pl.BlockSpec((1, BN, 128), lambda i: (1, i, 0)),
        pl.BlockSpec((H, H), lambda i: (0, 0)),
        pl.BlockSpec((H, H), lambda i: (0, 0)),
        pl.BlockSpec((H, D), lambda i: (0, 0)),
        pl.BlockSpec((1, D), lambda i: (0, 0)),
    ],
    out_specs=pl.BlockSpec((BN, D), lambda i: (i, 0)),
    out_shape=jax.ShapeDtypeStruct((N, D), jnp.float32),
)


def _tcd_body(a0_ref, a1_ref, d0_ref, d1_ref, m0_ref, m1_ref, x_ref, o_ref):
    i = pl.program_id(0)
    agg = a0_ref[0] + a1_ref[0]
    deg = jnp.maximum(d0_ref[0][:, 0:1] + d1_ref[0][:, 0:1], 1.0)
    recon = agg / deg
    xb = x_ref[...]
    xn = xb / (jnp.sqrt(jnp.sum(xb * xb, axis=-1, keepdims=True)) + 1e-8)
    rn = recon / (jnp.sqrt(jnp.sum(recon * recon, axis=-1, keepdims=True)) + 1e-8)
    cos = jnp.sum(xn * rn, axis=-1)
    mcol = m0_ref[0][:, 0] + m1_ref[0][:, 0]
    part = jnp.sum(mcol * (1.0 - cos) ** 2) * (1.0 / NUM_MASK)

    @pl.when(i == 0)
    def _():
        o_ref[...] = jnp.zeros((1, 1), jnp.float32)

    o_ref[...] = o_ref[...] + part


_tcd = pl.pallas_call(
    _tcd_body,
    grid=(GRID,),
    in_specs=[
        pl.BlockSpec((1, BN, D), lambda i: (0, i, 0)),
        pl.BlockSpec((1, BN, D), lambda i: (1, i, 0)),
        pl.BlockSpec((1, BN, 128), lambda i: (0, i, 0)),
        pl.BlockSpec((1, BN, 128), lambda i: (1, i, 0)),
        pl.BlockSpec((1, BN, 128), lambda i: (0, i, 0)),
        pl.BlockSpec((1, BN, 128), lambda i: (1, i, 0)),
        pl.BlockSpec((BN, D), lambda i: (i, 0)),
    ],
    out_specs=pl.BlockSpec((1, 1), lambda i: (0, 0)),
    out_shape=jax.ShapeDtypeStruct((1, 1), jnp.float32),
)


def kernel(x, edge_index, mask_nodes, enc_mask_token, W1, b1, W2, b2, W_e2d, Wd, bd):
    src = edge_index[0]
    dst = edge_index[1]
    src3 = jnp.concatenate(
        [src, jnp.arange(EPAD - E, dtype=jnp.int32) % N]).reshape(NW, CPT, CHUNK)
    dst3 = jnp.concatenate(
        [dst, N + (jnp.arange(EPAD - E, dtype=jnp.int32) % SINK_ROWS)]
    ).reshape(NW, CPT, CHUNK)
    mn3 = jnp.concatenate(
        [mask_nodes.astype(jnp.int32),
         N + (jnp.arange(MASKPAD - NUM_MASK, dtype=jnp.int32) % SINK_ROWS)]
    ).reshape(NW, MCPT, CHUNK)
    ones128 = jnp.ones((CHUNK, H), jnp.float32)
    zsink = jnp.zeros((NSINK, H), jnp.float32)
    b1r = b1.reshape(1, H)
    b2r = b2.reshape(1, H)
    bdr = bd.reshape(1, D)
    we1 = W_e2d[:H]
    we2 = W_e2d[H:]

    degp = _sc_deg_hist(dst3, ones128, zsink)
    maskp = _sc_mask_hist(mn3, ones128, zsink)
    z1 = _tca(x, maskp, maskp, enc_mask_token, W1, b1r)
    a1 = _sc_agg(z1, src3, dst3, zsink)
    h1, z2 = _tcb(a1, a1, degp, degp, W2, b2r)
    a2 = _sc_agg(z2, src3, dst3, zsink)
    z3 = _tcc(a2, a2, degp, degp, h1, maskp, maskp, we1, we2, Wd, bdr)
    a3 = _sc_agg(z3, src3, dst3, zsink)
    lossm = _tcd(a3, a3, degp, degp, maskp, maskp, x)
    return lossm[0, 0]
